# baseline (speedup 1.0000x reference)
import functools

import jax
import jax.numpy as jnp
from jax import lax
from jax.experimental import pallas as pl
from jax.experimental.pallas import tpu as pltpu
from jax.experimental.pallas import tpu_sc as plsc

N = 4096
MAX_LEN = 200
D_ITEM = 128
POOL = 100000
D_CTX = 64

NUM_CORES = 2
NUM_SUBCORES = 16
NUM_WORKERS = NUM_CORES * NUM_SUBCORES
SEQ_PER_W = N // NUM_WORKERS
LANES = 16

CHUNKS = ((0, 56), (56, 48), (104, 48), (152, 48))
SLOT = 64
IDX_W = SLOT * len(CHUNKS)

MASK_W = SEQ_PER_W * MAX_LEN
CTX_W = SEQ_PER_W * D_CTX
MASK_PAD_CHUNKS = -(-MAX_LEN // LANES)


_GATHER_DNUMS = lax.GatherDimensionNumbers(
    offset_dims=(), collapsed_slice_dims=(0,), start_index_map=(0,))


def _splat(vec, lane):
    idx = jnp.full((LANES, 1), lane, jnp.int32)
    return lax.gather(vec, idx, _GATHER_DNUMS, slice_sizes=(1,),
                      mode=lax.GatherScatterMode.PROMISE_IN_BOUNDS)


def _sampler_mesh_kernel():
    mesh = plsc.VectorSubcoreMesh(core_axis_name="c", subcore_axis_name="s")

    @functools.partial(
        pl.kernel,
        mesh=mesh,
        out_type=(
            jax.ShapeDtypeStruct((N, MAX_LEN, D_ITEM), jnp.float32),
            jax.ShapeDtypeStruct((N * MAX_LEN,), jnp.float32),
            jax.ShapeDtypeStruct((N * D_CTX,), jnp.float32),
        ),
        scratch_types=[
            pltpu.VMEM((SEQ_PER_W, IDX_W), jnp.int32),
            pltpu.VMEM((SEQ_PER_W + LANES,), jnp.int32),
            pltpu.VMEM((MASK_W + LANES,), jnp.float32),
            pltpu.VMEM((CTX_W,), jnp.float32),
            pltpu.VMEM((MAX_LEN, D_ITEM), jnp.float32),
            pltpu.VMEM((MAX_LEN, D_ITEM), jnp.float32),
            pltpu.SemaphoreType.DMA,
            pltpu.SemaphoreType.DMA,
            pltpu.SemaphoreType.DMA,
            pltpu.SemaphoreType.DMA,
        ],
    )
    def body(universe, idx_hbm, sizes_hbm, ctx_hbm,
             items_out, mask_out, ctx_out,
             idx_all, sizes_v, mask_all, ctx_v, rows0, rows1,
             gsem0, gsem1, wsem0, wsem1):
        rows = (rows0, rows1)
        gsems = (gsem0, gsem1)
        wsems = (wsem0, wsem1)
        c = lax.axis_index("c")
        s = lax.axis_index("s")
        wid = s * NUM_CORES + c
        base = wid * SEQ_PER_W

        pltpu.sync_copy(idx_hbm.at[pl.ds(base, SEQ_PER_W)], idx_all)
        pltpu.sync_copy(sizes_hbm.at[pl.ds(base, SEQ_PER_W)],
                        sizes_v.at[pl.ds(0, SEQ_PER_W)])
        pltpu.sync_copy(ctx_hbm.at[pl.ds(base * D_CTX, CTX_W)], ctx_v)

        def size_of(b):
            return sizes_v[pl.ds(b, LANES)][0]

        def issue_gathers(b, buf, gsem):
            s_sc = size_of(b)
            for j, (start, length) in enumerate(CHUNKS):
                @pl.when(jnp.int32(start) < s_sc)
                def _issue(j=j, start=start, length=length):
                    pltpu.async_copy(
                        universe.at[idx_all.at[b, pl.ds(j * SLOT, length)]],
                        buf.at[pl.ds(start, length)], gsem)

        def wait_gathers(b, buf, gsem):
            s_sc = size_of(b)
            for j, (start, length) in enumerate(CHUNKS):
                @pl.when(jnp.int32(start) < s_sc)
                def _drain(j=j, start=start, length=length):
                    pltpu.make_async_copy(
                        universe.at[idx_all.at[b, pl.ds(j * SLOT, length)]],
                        buf.at[pl.ds(start, length)], gsem).wait()

        def issue_write(b, buf, wsem):
            pltpu.async_copy(buf, items_out.at[base + b], wsem)

        def wait_write(b, buf, wsem):
            pltpu.make_async_copy(buf, items_out.at[base + b], wsem).wait()

        def mask_body(b, carry):
            schunk = sizes_v[pl.ds(jnp.bitwise_and(b, -LANES), LANES)]
            svec = _splat(schunk, jnp.bitwise_and(b, LANES - 1))
            for k in range(MASK_PAD_CHUNKS):
                pos = lax.iota(jnp.int32, LANES) + (k * LANES)
                mask_all[pl.ds(b * MAX_LEN + k * LANES, LANES)] = jnp.where(
                    pos < svec, 1.0, 0.0)
            return carry

        lax.fori_loop(0, SEQ_PER_W, mask_body, 0)

        def ctx_body(t, carry):
            v = ctx_v[pl.ds(t * LANES, LANES)]
            ctx_v[pl.ds(t * LANES, LANES)] = jnp.minimum(
                jnp.maximum(v, -3.0), 3.0)
            return carry

        lax.fori_loop(0, CTX_W // LANES, ctx_body, 0)

        zeros16 = jnp.zeros((LANES,), jnp.float32)

        issue_gathers(0, rows[0], gsems[0])

        def pair_body(t, carry):
            for p in range(2):
                b = 2 * t + p
                q = 1 - p

                @pl.when(b <= SEQ_PER_W - 2)
                def _next(b=b, q=q):
                    @pl.when(b >= 1)
                    def _reuse(b=b, q=q):
                        wait_write(b - 1, rows[q], wsems[q])
                    issue_gathers(b + 1, rows[q], gsems[q])

                wait_gathers(b, rows[p], gsems[p])

                s_sc = size_of(b)

                def zero_row(r, zcarry, p=p):
                    for k in range(D_ITEM // LANES):
                        rows[p][r, pl.ds(k * LANES, LANES)] = zeros16
                    return zcarry

                lax.fori_loop(s_sc, MAX_LEN, zero_row, 0)
                issue_write(b, rows[p], wsems[p])
            return carry

        lax.fori_loop(0, SEQ_PER_W // 2, pair_body, 0)

        wait_write(SEQ_PER_W - 2, rows[0], wsems[0])
        wait_write(SEQ_PER_W - 1, rows[1], wsems[1])

        pltpu.sync_copy(mask_all.at[pl.ds(0, MASK_W)],
                        mask_out.at[pl.ds(base * MAX_LEN, MASK_W)])
        pltpu.sync_copy(ctx_v, ctx_out.at[pl.ds(base * D_CTX, CTX_W)])

    return body


_SAMPLER = _sampler_mesh_kernel()


def kernel(item_universe, context, chosen_idx, sizes):
    parts = [
        jnp.pad(chosen_idx[:, start:start + length],
                ((0, 0), (0, SLOT - length)))
        for start, length in CHUNKS
    ]
    idx_packed = jnp.concatenate(parts, axis=1)
    items, mask_flat, ctx_flat = _SAMPLER(
        item_universe, idx_packed, sizes, context.reshape(-1))
    return (items, mask_flat.reshape(N, MAX_LEN),
            ctx_flat.reshape(N, D_CTX))


# speedup vs baseline: 1.0106x; 1.0106x over previous
import functools

import jax
import jax.numpy as jnp
from jax import lax
from jax.experimental import pallas as pl
from jax.experimental.pallas import tpu as pltpu
from jax.experimental.pallas import tpu_sc as plsc

N = 4096
MAX_LEN = 200
D_ITEM = 128
POOL = 100000
D_CTX = 64

NUM_CORES = 2
NUM_SUBCORES = 16
NUM_WORKERS = NUM_CORES * NUM_SUBCORES
SEQ_PER_W = N // NUM_WORKERS
LANES = 16

CHUNKS = ((0, 56), (56, 48), (104, 48), (152, 48))
SLOT = 64
IDX_W = SLOT * len(CHUNKS)

MASK_W = SEQ_PER_W * MAX_LEN
CTX_W = SEQ_PER_W * D_CTX
MASK_PAD_CHUNKS = -(-MAX_LEN // LANES)


_GATHER_DNUMS = lax.GatherDimensionNumbers(
    offset_dims=(), collapsed_slice_dims=(0,), start_index_map=(0,))


def _splat(vec, lane):
    idx = jnp.full((LANES, 1), lane, jnp.int32)
    return lax.gather(vec, idx, _GATHER_DNUMS, slice_sizes=(1,),
                      mode=lax.GatherScatterMode.PROMISE_IN_BOUNDS)


def _sampler_mesh_kernel():
    mesh = plsc.VectorSubcoreMesh(core_axis_name="c", subcore_axis_name="s")

    @functools.partial(
        pl.kernel,
        mesh=mesh,
        out_type=(
            jax.ShapeDtypeStruct((N, MAX_LEN, D_ITEM), jnp.float32),
            jax.ShapeDtypeStruct((N * MAX_LEN,), jnp.float32),
            jax.ShapeDtypeStruct((N * D_CTX,), jnp.float32),
        ),
        scratch_types=[
            pltpu.VMEM((SEQ_PER_W, IDX_W), jnp.int32),
            pltpu.VMEM((SEQ_PER_W + LANES,), jnp.int32),
            pltpu.VMEM((MASK_W + LANES,), jnp.float32),
            pltpu.VMEM((CTX_W,), jnp.float32),
            pltpu.VMEM((MAX_LEN, D_ITEM), jnp.float32),
            pltpu.VMEM((MAX_LEN, D_ITEM), jnp.float32),
            pltpu.SemaphoreType.DMA,
            pltpu.SemaphoreType.DMA,
            pltpu.SemaphoreType.DMA,
            pltpu.SemaphoreType.DMA,
            pltpu.SemaphoreType.DMA,
        ],
    )
    def body(universe, idx_hbm, sizes_hbm, ctx_hbm,
             items_out, mask_out, ctx_out,
             idx_all, sizes_v, mask_all, ctx_v, rows0, rows1,
             gsem0, gsem1, wsem0, wsem1, msem):
        rows = (rows0, rows1)
        gsems = (gsem0, gsem1)
        wsems = (wsem0, wsem1)
        c = lax.axis_index("c")
        s = lax.axis_index("s")
        wid = s * NUM_CORES + c
        base = wid * SEQ_PER_W

        pltpu.sync_copy(idx_hbm.at[pl.ds(base, SEQ_PER_W)], idx_all)
        pltpu.sync_copy(sizes_hbm.at[pl.ds(base, SEQ_PER_W)],
                        sizes_v.at[pl.ds(0, SEQ_PER_W)])
        pltpu.sync_copy(ctx_hbm.at[pl.ds(base * D_CTX, CTX_W)], ctx_v)

        def size_of(b):
            return sizes_v[pl.ds(b, LANES)][0]

        def issue_gathers(b, buf, gsem):
            s_sc = size_of(b)
            for j, (start, length) in enumerate(CHUNKS):
                @pl.when(jnp.int32(start) < s_sc)
                def _issue(j=j, start=start, length=length):
                    pltpu.async_copy(
                        universe.at[idx_all.at[b, pl.ds(j * SLOT, length)]],
                        buf.at[pl.ds(start, length)], gsem)

        def wait_gathers(b, buf, gsem):
            s_sc = size_of(b)
            for j, (start, length) in enumerate(CHUNKS):
                @pl.when(jnp.int32(start) < s_sc)
                def _drain(j=j, start=start, length=length):
                    pltpu.make_async_copy(
                        universe.at[idx_all.at[b, pl.ds(j * SLOT, length)]],
                        buf.at[pl.ds(start, length)], gsem).wait()

        def issue_write(b, buf, wsem):
            pltpu.async_copy(buf, items_out.at[base + b], wsem)

        def wait_write(b, buf, wsem):
            pltpu.make_async_copy(buf, items_out.at[base + b], wsem).wait()

        issue_gathers(0, rows[0], gsems[0])
        issue_gathers(1, rows[1], gsems[1])

        def mask_body(b, carry):
            schunk = sizes_v[pl.ds(jnp.bitwise_and(b, -LANES), LANES)]
            svec = _splat(schunk, jnp.bitwise_and(b, LANES - 1))
            for k in range(MASK_PAD_CHUNKS):
                pos = lax.iota(jnp.int32, LANES) + (k * LANES)
                mask_all[pl.ds(b * MAX_LEN + k * LANES, LANES)] = jnp.where(
                    pos < svec, 1.0, 0.0)
            return carry

        lax.fori_loop(0, SEQ_PER_W, mask_body, 0)

        def ctx_body(t, carry):
            v = ctx_v[pl.ds(t * LANES, LANES)]
            ctx_v[pl.ds(t * LANES, LANES)] = jnp.minimum(
                jnp.maximum(v, -3.0), 3.0)
            return carry

        lax.fori_loop(0, CTX_W // LANES, ctx_body, 0)

        pltpu.async_copy(mask_all.at[pl.ds(0, MASK_W)],
                         mask_out.at[pl.ds(base * MAX_LEN, MASK_W)], msem)
        pltpu.async_copy(ctx_v, ctx_out.at[pl.ds(base * D_CTX, CTX_W)], msem)

        zeros16 = jnp.zeros((LANES,), jnp.float32)

        def straddle_end(s_sc):
            end = jnp.int32(CHUNKS[-1][0] + CHUNKS[-1][1])
            for start, length in reversed(CHUNKS[:-1]):
                end = jnp.where(s_sc <= start + length,
                                jnp.int32(start + length), end)
            return end

        def pair_body(t, carry):
            hwm = list(carry)
            for p in range(2):
                b = 2 * t + p
                q = 1 - p

                @pl.when(jnp.logical_and(b >= 1, b <= SEQ_PER_W - 2))
                def _next(b=b, q=q):
                    wait_write(b - 1, rows[q], wsems[q])
                    issue_gathers(b + 1, rows[q], gsems[q])

                wait_gathers(b, rows[p], gsems[p])

                s_sc = size_of(b)

                def zero_row(r, zcarry, p=p):
                    for k in range(D_ITEM // LANES):
                        rows[p][r, pl.ds(k * LANES, LANES)] = zeros16
                    return zcarry

                zend = jnp.maximum(straddle_end(s_sc), hwm[p])
                lax.fori_loop(s_sc, zend, zero_row, 0)
                issue_write(b, rows[p], wsems[p])
                hwm[p] = s_sc
            return tuple(hwm)

        lax.fori_loop(0, SEQ_PER_W // 2, pair_body,
                      (jnp.int32(MAX_LEN), jnp.int32(MAX_LEN)))

        wait_write(SEQ_PER_W - 2, rows[0], wsems[0])
        wait_write(SEQ_PER_W - 1, rows[1], wsems[1])
        pltpu.make_async_copy(
            mask_all.at[pl.ds(0, MASK_W)],
            mask_out.at[pl.ds(base * MAX_LEN, MASK_W)], msem).wait()
        pltpu.make_async_copy(
            ctx_v, ctx_out.at[pl.ds(base * D_CTX, CTX_W)], msem).wait()

    return body


_SAMPLER = _sampler_mesh_kernel()


def kernel(item_universe, context, chosen_idx, sizes):
    parts = [
        jnp.pad(chosen_idx[:, start:start + length],
                ((0, 0), (0, SLOT - length)))
        for start, length in CHUNKS
    ]
    idx_packed = jnp.concatenate(parts, axis=1)
    items, mask_flat, ctx_flat = _SAMPLER(
        item_universe, idx_packed, sizes, context.reshape(-1))
    return (items, mask_flat.reshape(N, MAX_LEN),
            ctx_flat.reshape(N, D_CTX))


# speedup vs baseline: 1.0395x; 1.0287x over previous
import functools

import jax
import jax.numpy as jnp
from jax import lax
from jax.experimental import pallas as pl
from jax.experimental.pallas import tpu as pltpu
from jax.experimental.pallas import tpu_sc as plsc

N = 4096
MAX_LEN = 200
D_ITEM = 128
POOL = 100000
D_CTX = 64

NUM_CORES = 2
NUM_SUBCORES = 16
NUM_WORKERS = NUM_CORES * NUM_SUBCORES
SEQ_PER_W = N // NUM_WORKERS
LANES = 16

CHUNKS = ((0, 64), (64, 64), (128, 40), (168, 32))
IDX_W = 256

MASK_W = SEQ_PER_W * MAX_LEN
CTX_W = SEQ_PER_W * D_CTX
MASK_PAD_CHUNKS = -(-MAX_LEN // LANES)


_GATHER_DNUMS = lax.GatherDimensionNumbers(
    offset_dims=(), collapsed_slice_dims=(0,), start_index_map=(0,))


def _splat(vec, lane):
    idx = jnp.full((LANES, 1), lane, jnp.int32)
    return lax.gather(vec, idx, _GATHER_DNUMS, slice_sizes=(1,),
                      mode=lax.GatherScatterMode.PROMISE_IN_BOUNDS)


def _sampler_mesh_kernel():
    mesh = plsc.VectorSubcoreMesh(core_axis_name="c", subcore_axis_name="s")

    @functools.partial(
        pl.kernel,
        mesh=mesh,
        out_type=(
            jax.ShapeDtypeStruct((N, MAX_LEN, D_ITEM), jnp.float32),
            jax.ShapeDtypeStruct((N * MAX_LEN,), jnp.float32),
            jax.ShapeDtypeStruct((N * D_CTX,), jnp.float32),
        ),
        scratch_types=[
            pltpu.VMEM((SEQ_PER_W, IDX_W), jnp.int32),
            pltpu.VMEM((SEQ_PER_W + LANES,), jnp.int32),
            pltpu.VMEM((MASK_W + LANES,), jnp.float32),
            pltpu.VMEM((CTX_W,), jnp.float32),
            pltpu.VMEM((MAX_LEN, D_ITEM), jnp.float32),
            pltpu.VMEM((MAX_LEN, D_ITEM), jnp.float32),
            pltpu.SemaphoreType.DMA,
            pltpu.SemaphoreType.DMA,
            pltpu.SemaphoreType.DMA,
            pltpu.SemaphoreType.DMA,
            pltpu.SemaphoreType.DMA,
        ],
    )
    def body(universe, idx_hbm, sizes_hbm, ctx_hbm,
             items_out, mask_out, ctx_out,
             idx_all, sizes_v, mask_all, ctx_v, rows0, rows1,
             gsem0, gsem1, wsem0, wsem1, msem):
        rows = (rows0, rows1)
        gsems = (gsem0, gsem1)
        wsems = (wsem0, wsem1)
        c = lax.axis_index("c")
        s = lax.axis_index("s")
        wid = s * NUM_CORES + c
        base = wid * SEQ_PER_W

        pltpu.sync_copy(idx_hbm.at[pl.ds(base, SEQ_PER_W)], idx_all)
        pltpu.sync_copy(sizes_hbm.at[pl.ds(base, SEQ_PER_W)],
                        sizes_v.at[pl.ds(0, SEQ_PER_W)])
        pltpu.sync_copy(ctx_hbm.at[pl.ds(base * D_CTX, CTX_W)], ctx_v)

        def size_of(b):
            return sizes_v[pl.ds(b, LANES)][0]

        def issue_gathers(b, buf, gsem):
            s_sc = size_of(b)
            for j, (start, length) in enumerate(CHUNKS):
                @pl.when(jnp.int32(start) < s_sc)
                def _issue(j=j, start=start, length=length):
                    pltpu.async_copy(
                        universe.at[idx_all.at[b, pl.ds(start, length)]],
                        buf.at[pl.ds(start, length)], gsem)

        def wait_gathers(b, buf, gsem):
            s_sc = size_of(b)
            for j, (start, length) in enumerate(CHUNKS):
                @pl.when(jnp.int32(start) < s_sc)
                def _drain(j=j, start=start, length=length):
                    pltpu.make_async_copy(
                        universe.at[idx_all.at[b, pl.ds(start, length)]],
                        buf.at[pl.ds(start, length)], gsem).wait()

        def issue_write(b, buf, wsem):
            pltpu.async_copy(buf, items_out.at[base + b], wsem)

        def wait_write(b, buf, wsem):
            pltpu.make_async_copy(buf, items_out.at[base + b], wsem).wait()

        issue_gathers(0, rows[0], gsems[0])
        issue_gathers(1, rows[1], gsems[1])

        def mask_body(b, carry):
            schunk = sizes_v[pl.ds(jnp.bitwise_and(b, -LANES), LANES)]
            svec = _splat(schunk, jnp.bitwise_and(b, LANES - 1))
            for k in range(MASK_PAD_CHUNKS):
                pos = lax.iota(jnp.int32, LANES) + (k * LANES)
                mask_all[pl.ds(b * MAX_LEN + k * LANES, LANES)] = jnp.where(
                    pos < svec, 1.0, 0.0)
            return carry

        lax.fori_loop(0, SEQ_PER_W, mask_body, 0)

        def ctx_body(t, carry):
            v = ctx_v[pl.ds(t * LANES, LANES)]
            ctx_v[pl.ds(t * LANES, LANES)] = jnp.minimum(
                jnp.maximum(v, -3.0), 3.0)
            return carry

        lax.fori_loop(0, CTX_W // LANES, ctx_body, 0)

        pltpu.async_copy(mask_all.at[pl.ds(0, MASK_W)],
                         mask_out.at[pl.ds(base * MAX_LEN, MASK_W)], msem)
        pltpu.async_copy(ctx_v, ctx_out.at[pl.ds(base * D_CTX, CTX_W)], msem)

        zeros16 = jnp.zeros((LANES,), jnp.float32)

        def straddle_end(s_sc):
            end = jnp.int32(CHUNKS[-1][0] + CHUNKS[-1][1])
            for start, length in reversed(CHUNKS[:-1]):
                end = jnp.where(s_sc <= start + length,
                                jnp.int32(start + length), end)
            return end

        def pair_body(t, carry):
            hwm = list(carry)
            for p in range(2):
                b = 2 * t + p
                q = 1 - p

                @pl.when(jnp.logical_and(b >= 1, b <= SEQ_PER_W - 2))
                def _next(b=b, q=q):
                    wait_write(b - 1, rows[q], wsems[q])
                    issue_gathers(b + 1, rows[q], gsems[q])

                wait_gathers(b, rows[p], gsems[p])

                s_sc = size_of(b)

                def zero_row(r, zcarry, p=p):
                    for k in range(D_ITEM // LANES):
                        rows[p][r, pl.ds(k * LANES, LANES)] = zeros16
                    return zcarry

                zend = jnp.maximum(straddle_end(s_sc), hwm[p])
                lax.fori_loop(s_sc, zend, zero_row, 0)
                issue_write(b, rows[p], wsems[p])
                hwm[p] = s_sc
            return tuple(hwm)

        lax.fori_loop(0, SEQ_PER_W // 2, pair_body,
                      (jnp.int32(MAX_LEN), jnp.int32(MAX_LEN)))

        wait_write(SEQ_PER_W - 2, rows[0], wsems[0])
        wait_write(SEQ_PER_W - 1, rows[1], wsems[1])
        pltpu.make_async_copy(
            mask_all.at[pl.ds(0, MASK_W)],
            mask_out.at[pl.ds(base * MAX_LEN, MASK_W)], msem).wait()
        pltpu.make_async_copy(
            ctx_v, ctx_out.at[pl.ds(base * D_CTX, CTX_W)], msem).wait()

    return body


_SAMPLER = _sampler_mesh_kernel()


def kernel(item_universe, context, chosen_idx, sizes):
    idx_padded = jnp.pad(chosen_idx, ((0, 0), (0, IDX_W - MAX_LEN)))
    items, mask_flat, ctx_flat = _SAMPLER(
        item_universe, idx_padded, sizes, context.reshape(-1))
    return (items, mask_flat.reshape(N, MAX_LEN),
            ctx_flat.reshape(N, D_CTX))


# speedup vs baseline: 1.0468x; 1.0070x over previous
import functools

import jax
import jax.numpy as jnp
from jax import lax
from jax.experimental import pallas as pl
from jax.experimental.pallas import tpu as pltpu
from jax.experimental.pallas import tpu_sc as plsc

N = 4096
MAX_LEN = 200
D_ITEM = 128
POOL = 100000
D_CTX = 64

NUM_CORES = 2
NUM_SUBCORES = 16
NUM_WORKERS = NUM_CORES * NUM_SUBCORES
SEQ_PER_W = N // NUM_WORKERS
LANES = 16

CHUNKS = ((0, 64), (64, 64), (128, 40), (168, 32))
IDX_W = 256

MASK_W = SEQ_PER_W * MAX_LEN
CTX_W = SEQ_PER_W * D_CTX
MASK_PAD_CHUNKS = -(-MAX_LEN // LANES)


_GATHER_DNUMS = lax.GatherDimensionNumbers(
    offset_dims=(), collapsed_slice_dims=(0,), start_index_map=(0,))


def _splat(vec, lane):
    idx = jnp.full((LANES, 1), lane, jnp.int32)
    return lax.gather(vec, idx, _GATHER_DNUMS, slice_sizes=(1,),
                      mode=lax.GatherScatterMode.PROMISE_IN_BOUNDS)


def _sampler_mesh_kernel():
    mesh = plsc.VectorSubcoreMesh(core_axis_name="c", subcore_axis_name="s")

    @functools.partial(
        pl.kernel,
        mesh=mesh,
        out_type=(
            jax.ShapeDtypeStruct((N, MAX_LEN, D_ITEM), jnp.float32),
            jax.ShapeDtypeStruct((N * MAX_LEN,), jnp.float32),
            jax.ShapeDtypeStruct((N * D_CTX,), jnp.float32),
        ),
        scratch_types=[
            pltpu.VMEM((SEQ_PER_W, MAX_LEN), jnp.int32),
            pltpu.VMEM((SEQ_PER_W + LANES,), jnp.int32),
            pltpu.VMEM((MASK_W + LANES,), jnp.float32),
            pltpu.VMEM((CTX_W,), jnp.float32),
            pltpu.VMEM((MAX_LEN, D_ITEM), jnp.float32),
            pltpu.VMEM((MAX_LEN, D_ITEM), jnp.float32),
            pltpu.SemaphoreType.DMA,
            pltpu.SemaphoreType.DMA,
            pltpu.SemaphoreType.DMA,
            pltpu.SemaphoreType.DMA,
            pltpu.SemaphoreType.DMA,
        ],
    )
    def body(universe, idx_hbm, sizes_hbm, ctx_hbm,
             items_out, mask_out, ctx_out,
             idx_all, sizes_v, mask_all, ctx_v, rows0, rows1,
             gsem0, gsem1, wsem0, wsem1, msem):
        rows = (rows0, rows1)
        gsems = (gsem0, gsem1)
        wsems = (wsem0, wsem1)
        c = lax.axis_index("c")
        s = lax.axis_index("s")
        wid = s * NUM_CORES + c
        base = wid * SEQ_PER_W

        pltpu.sync_copy(idx_hbm.at[pl.ds(base, SEQ_PER_W)], idx_all)
        pltpu.sync_copy(sizes_hbm.at[pl.ds(base, SEQ_PER_W)],
                        sizes_v.at[pl.ds(0, SEQ_PER_W)])
        pltpu.sync_copy(ctx_hbm.at[pl.ds(base * D_CTX, CTX_W)], ctx_v)

        def size_of(b):
            return sizes_v[pl.ds(b, LANES)][0]

        def issue_gathers(b, buf, gsem):
            s_sc = size_of(b)
            for j, (start, length) in enumerate(CHUNKS):
                @pl.when(jnp.int32(start) < s_sc)
                def _issue(j=j, start=start, length=length):
                    pltpu.async_copy(
                        universe.at[idx_all.at[b, pl.ds(start, length)]],
                        buf.at[pl.ds(start, length)], gsem)

        def wait_gathers(b, buf, gsem):
            s_sc = size_of(b)
            for j, (start, length) in enumerate(CHUNKS):
                @pl.when(jnp.int32(start) < s_sc)
                def _drain(j=j, start=start, length=length):
                    pltpu.make_async_copy(
                        universe.at[idx_all.at[b, pl.ds(start, length)]],
                        buf.at[pl.ds(start, length)], gsem).wait()

        def issue_write(b, buf, wsem):
            pltpu.async_copy(buf, items_out.at[base + b], wsem)

        def wait_write(b, buf, wsem):
            pltpu.make_async_copy(buf, items_out.at[base + b], wsem).wait()

        issue_gathers(0, rows[0], gsems[0])
        issue_gathers(1, rows[1], gsems[1])

        def mask_body(b, carry):
            schunk = sizes_v[pl.ds(jnp.bitwise_and(b, -LANES), LANES)]
            svec = _splat(schunk, jnp.bitwise_and(b, LANES - 1))
            for k in range(MASK_PAD_CHUNKS):
                pos = lax.iota(jnp.int32, LANES) + (k * LANES)
                mask_all[pl.ds(b * MAX_LEN + k * LANES, LANES)] = jnp.where(
                    pos < svec, 1.0, 0.0)
            return carry

        lax.fori_loop(0, SEQ_PER_W, mask_body, 0)

        def ctx_body(t, carry):
            v = ctx_v[pl.ds(t * LANES, LANES)]
            ctx_v[pl.ds(t * LANES, LANES)] = jnp.minimum(
                jnp.maximum(v, -3.0), 3.0)
            return carry

        lax.fori_loop(0, CTX_W // LANES, ctx_body, 0)

        pltpu.async_copy(mask_all.at[pl.ds(0, MASK_W)],
                         mask_out.at[pl.ds(base * MAX_LEN, MASK_W)], msem)
        pltpu.async_copy(ctx_v, ctx_out.at[pl.ds(base * D_CTX, CTX_W)], msem)

        zeros16 = jnp.zeros((LANES,), jnp.float32)

        def straddle_end(s_sc):
            end = jnp.int32(CHUNKS[-1][0] + CHUNKS[-1][1])
            for start, length in reversed(CHUNKS[:-1]):
                end = jnp.where(s_sc <= start + length,
                                jnp.int32(start + length), end)
            return end

        def pair_body(t, carry):
            hwm = list(carry)
            for p in range(2):
                b = 2 * t + p
                q = 1 - p

                @pl.when(jnp.logical_and(b >= 1, b <= SEQ_PER_W - 2))
                def _next(b=b, q=q):
                    wait_write(b - 1, rows[q], wsems[q])
                    issue_gathers(b + 1, rows[q], gsems[q])

                wait_gathers(b, rows[p], gsems[p])

                s_sc = size_of(b)

                def zero_row(r, zcarry, p=p):
                    for k in range(D_ITEM // LANES):
                        rows[p][r, pl.ds(k * LANES, LANES)] = zeros16
                    return zcarry

                zend = jnp.maximum(straddle_end(s_sc), hwm[p])
                lax.fori_loop(s_sc, zend, zero_row, 0)
                issue_write(b, rows[p], wsems[p])
                hwm[p] = s_sc
            return tuple(hwm)

        lax.fori_loop(0, SEQ_PER_W // 2, pair_body,
                      (jnp.int32(MAX_LEN), jnp.int32(MAX_LEN)))

        wait_write(SEQ_PER_W - 2, rows[0], wsems[0])
        wait_write(SEQ_PER_W - 1, rows[1], wsems[1])
        pltpu.make_async_copy(
            mask_all.at[pl.ds(0, MASK_W)],
            mask_out.at[pl.ds(base * MAX_LEN, MASK_W)], msem).wait()
        pltpu.make_async_copy(
            ctx_v, ctx_out.at[pl.ds(base * D_CTX, CTX_W)], msem).wait()

    return body


_SAMPLER = _sampler_mesh_kernel()


def kernel(item_universe, context, chosen_idx, sizes):
    items, mask_flat, ctx_flat = _SAMPLER(
        item_universe, chosen_idx, sizes, context.reshape(-1))
    return (items, mask_flat.reshape(N, MAX_LEN),
            ctx_flat.reshape(N, D_CTX))
